# baseline (speedup 1.0000x reference)
import jax
import jax.numpy as jnp
from jax import lax
from jax.experimental import pallas as pl
from jax.experimental.pallas import tpu as pltpu

_TILES_PER_STREAM = 1024
_NSTREAMS = 4


def _mlp_block(x, w1, b1, w2, b2):
    h = lax.dot_general(
        w1, x,
        dimension_numbers=(((1,), (1,)), ((), ())),
        preferred_element_type=jnp.float32)
    h = jnp.maximum(h + b1.astype(jnp.float32), 0.0)
    y = jnp.dot(w2, h.astype(w2.dtype), preferred_element_type=jnp.float32)
    return y + b2.astype(jnp.float32)


def _mlp_kernel(*refs):
    xrefs = refs[:_NSTREAMS]
    w1_ref, b1_ref, w2_ref, b2_ref, o_ref = refs[_NSTREAMS:]
    w1 = w1_ref[...]
    b1 = b1_ref[...]
    w2 = w2_ref[...]
    b2 = b2_ref[...]
    T, _, in_dim = xrefs[0].shape
    for k, x_ref in enumerate(xrefs):
        x = x_ref[...].reshape(T * 8, in_dim)
        y = _mlp_block(x, w1, b1, w2, b2)
        o_ref[:, 8 * T * k: 8 * T * (k + 1)] = y.astype(o_ref.dtype)


def kernel(x, w1, b1, w2, b2):
    B, in_dim = x.shape
    hid = w1.shape[0]
    out_dim = w2.shape[0]

    b1c = b1.reshape(hid, 1)
    b2c = b2.reshape(out_dim, 1)

    ntiles = B // 8
    x3 = x.reshape(ntiles, 8, in_dim)
    ns = _NSTREAMS
    T = min(ntiles // ns, _TILES_PER_STREAM)
    grid = (ntiles // (ns * T),)

    x_specs = [
        pl.BlockSpec((T, 8, in_dim),
                     (lambda k: (lambda i: (ns * i + k, 0, 0)))(k))
        for k in range(ns)
    ]

    yt = pl.pallas_call(
        _mlp_kernel,
        out_shape=jax.ShapeDtypeStruct((out_dim, B), x.dtype),
        grid=grid,
        in_specs=x_specs + [
            pl.BlockSpec((hid, in_dim), lambda i: (0, 0)),
            pl.BlockSpec((hid, 1), lambda i: (0, 0)),
            pl.BlockSpec((out_dim, hid), lambda i: (0, 0)),
            pl.BlockSpec((out_dim, 1), lambda i: (0, 0)),
        ],
        out_specs=pl.BlockSpec((out_dim, ns * 8 * T), lambda i: (0, i)),
        compiler_params=pltpu.CompilerParams(
            dimension_semantics=("parallel",),
            vmem_limit_bytes=60 << 20,
        ),
    )(*([x3] * ns), w1, b1c, w2, b2c)

    return yt.T
